# baseline (speedup 1.0000x reference)
import functools

import jax
import jax.numpy as jnp
from jax import lax
from jax.experimental import pallas as pl
from jax.experimental.pallas import tpu as pltpu
from jax.experimental.pallas import tpu_sc as plsc

_BATCH = 16384
_N_FIELDS = 26
_B = _BATCH * _N_FIELDS
_D = 32
_V = 1000000
_NC = 2
_NS = 16
_NW = _NC * _NS
_CBF = _V // 128
_TAIL = _V - _CBF * 128
_TPW = (_CBF + _NW - 1) // _NW
_BPW = _BATCH // _NW
_NGRP = _N_FIELDS * 4


def _make_transpose():
    mesh = plsc.VectorSubcoreMesh(core_axis_name="c", subcore_axis_name="s")

    @functools.partial(
        pl.kernel,
        mesh=mesh,
        out_type=jax.ShapeDtypeStruct((_V // 4, 128), jnp.float32),
        compiler_params=pltpu.CompilerParams(needs_layout_passes=False),
        scratch_types=[
            pltpu.VMEM((32, 128), jnp.float32),
            pltpu.VMEM((32, 128), jnp.float32),
            pltpu.VMEM((32, 128), jnp.float32),
            pltpu.VMEM((32, 128), jnp.float32),
            pltpu.SemaphoreType.DMA,
            pltpu.SemaphoreType.DMA,
            pltpu.SemaphoreType.DMA,
            pltpu.SemaphoreType.DMA,
        ],
    )
    def tr(tblT_hbm, tail_hbm, s128_hbm, in0, in1, ot0, ot1, gi0, gi1, go0, go1):
        wid = lax.axis_index("s") * _NC + lax.axis_index("c")
        ins = (in0, in1)
        ots = (ot0, ot1)
        gis = (gi0, gi1)
        gos = (go0, go1)
        lane = lax.iota(jnp.int32, 16)

        def start_in(t, b):
            bid = t * _NW + wid
            @pl.when(bid < _CBF)
            def _():
                pltpu.async_copy(
                    tblT_hbm.at[:, pl.ds(bid * 128, 128)], ins[b], gis[b])

        def wait_in(b):
            pltpu.make_async_copy(
                tblT_hbm.at[:, pl.ds(0, 128)], ins[b], gis[b]).wait()

        def start_out(b, bid):
            pltpu.async_copy(
                ots[b], s128_hbm.at[pl.ds(bid * 32, 32), :], gos[b])

        def wait_out(b):
            pltpu.make_async_copy(
                ots[b], s128_hbm.at[pl.ds(0, 32), :], gos[b]).wait()

        def shuffle(b, nrow):
            @plsc.parallel_loop(0, nrow, unroll=1)
            def _(r):
                vs = []
                for m in range(8):
                    rows = lane + 16 * (m % 2)
                    cols = jnp.full((16,), jnp.int32(4) * r + m // 2, jnp.int32)
                    vs.append(plsc.load_gather(ins[b], [rows, cols]))
                for m in range(8):
                    ots[b][r, pl.ds(16 * m, 16)] = vs[m]

        def body(u, carry):
            for dt in (0, 1):
                t = 2 * u + dt
                b = dt
                bid = t * _NW + wid

                @pl.when(bid < _CBF)
                def _():
                    start_in(t + 1, 1 - b)
                    wait_in(b)
                    @pl.when(t >= 2)
                    def _():
                        wait_out(b)
                    shuffle(b, 32)
                    start_out(b, bid)
            return carry

        start_in(0, 0)
        lax.fori_loop(0, (_TPW + 1) // 2, body, 0)
        for dt in (0, 1):
            t = _TPW - 2 + dt
            bid = t * _NW + wid
            @pl.when(bid < _CBF)
            def _():
                wait_out(t % 2)

        @pl.when(wid == 0)
        def _():
            pltpu.sync_copy(tail_hbm, ot0.at[pl.ds(0, 16), :])
            pltpu.sync_copy(ot0.at[pl.ds(0, 16), :],
                            s128_hbm.at[pl.ds(_CBF * 32, 16), :])

    return tr


def _make_gather():
    mesh = plsc.VectorSubcoreMesh(core_axis_name="c", subcore_axis_name="s")

    @functools.partial(
        pl.kernel,
        mesh=mesh,
        out_type=jax.ShapeDtypeStruct((_N_FIELDS, _D, _BATCH), jnp.float32),
        compiler_params=pltpu.CompilerParams(needs_layout_passes=False),
        scratch_types=[
            pltpu.VMEM((_N_FIELDS, _BPW), jnp.int32),
            pltpu.VMEM((128,), jnp.int32),
            pltpu.VMEM((128,), jnp.int32),
            pltpu.VMEM((128, 128), jnp.float32),
            pltpu.VMEM((128, 128), jnp.float32),
            pltpu.VMEM((32, 128), jnp.float32),
            pltpu.VMEM((32, 128), jnp.float32),
            pltpu.SemaphoreType.DMA,
            pltpu.SemaphoreType.DMA,
            pltpu.SemaphoreType.DMA,
            pltpu.SemaphoreType.DMA,
        ],
    )
    def ga(idxT_hbm, s128_hbm, out3_hbm, idx_v, ri0, ri1, rv0, rv1, ot0, ot1,
           gsem0, gsem1, osem0, osem1):
        wid = lax.axis_index("s") * _NC + lax.axis_index("c")
        b0 = wid * _BPW
        pltpu.sync_copy(idxT_hbm.at[:, pl.ds(b0, _BPW)], idx_v)
        ris = (ri0, ri1)
        rvs = (rv0, rv1)
        ots = (ot0, ot1)
        gsems = (gsem0, gsem1)
        osems = (osem0, osem1)
        lane = lax.iota(jnp.int32, 16)

        def prep_and_gather(g, b):
            f = g // 4
            tb = g % 4
            ivs = [idx_v[f, pl.ds(tb * 128 + 16 * m, 16)] for m in range(8)]
            for m in range(8):
                ris[b][pl.ds(16 * m, 16)] = ivs[m] >> 2
            pltpu.async_copy(s128_hbm.at[ris[b]], rvs[b], gsems[b])

        def wait_gather(b):
            pltpu.make_async_copy(s128_hbm.at[ris[b]], rvs[b], gsems[b]).wait()

        def extract(g, b):
            f = g // 4
            tb = g % 4
            avecs = []
            for m in range(8):
                iv = idx_v[f, pl.ds(tb * 128 + 16 * m, 16)]
                avecs.append((iv & 3) << 5)
            @plsc.parallel_loop(0, 32, unroll=1)
            def _(j):
                vs = []
                for m in range(8):
                    vs.append(plsc.load_gather(
                        rvs[b], [lane + 16 * m, avecs[m] + j]))
                for m in range(8):
                    ots[b][j, pl.ds(16 * m, 16)] = vs[m]

        def start_out(g, b):
            f = g // 4
            tb = g % 4
            pltpu.async_copy(
                ots[b], out3_hbm.at[f, :, pl.ds((4 * wid + tb) * 128, 128)],
                osems[b])

        def wait_out(b):
            pltpu.make_async_copy(
                ots[b], out3_hbm.at[0, :, pl.ds(0, 128)], osems[b]).wait()

        def body(u, carry):
            for dt in (0, 1):
                g = 2 * u + dt
                b = dt
                @pl.when(g + 1 < _NGRP)
                def _():
                    prep_and_gather(g + 1, 1 - b)
                wait_gather(b)
                @pl.when(g >= 2)
                def _():
                    wait_out(b)
                extract(g, b)
                start_out(g, b)
            return carry

        prep_and_gather(0, 0)
        lax.fori_loop(0, _NGRP // 2, body, 0)
        wait_out(0)
        wait_out(1)

    return ga


_tr = _make_transpose()
_ga = _make_gather()


def kernel(x, table):
    tail = table[_CBF * 128:].reshape(16, 128)
    s128 = _tr(table.T, tail)
    out3 = _ga(x.T, s128)
    return out3.transpose(2, 0, 1)


# speedup vs baseline: 1.0377x; 1.0377x over previous
import functools

import jax
import jax.numpy as jnp
from jax import lax
from jax.experimental import pallas as pl
from jax.experimental.pallas import tpu as pltpu
from jax.experimental.pallas import tpu_sc as plsc

_BATCH = 16384
_N_FIELDS = 26
_B = _BATCH * _N_FIELDS
_D = 32
_V = 1000000
_NC = 2
_NS = 16
_NW = _NC * _NS
_CBF = _V // 128
_TAIL = _V - _CBF * 128
_TPW = (_CBF + _NW - 1) // _NW
_BPW = _BATCH // _NW
_NGRP = _N_FIELDS * 4


def _make_transpose():
    mesh = plsc.VectorSubcoreMesh(core_axis_name="c", subcore_axis_name="s")

    @functools.partial(
        pl.kernel,
        mesh=mesh,
        out_type=jax.ShapeDtypeStruct((_V // 4, 128), jnp.float32),
        compiler_params=pltpu.CompilerParams(needs_layout_passes=False),
        scratch_types=[
            pltpu.VMEM((32, 128), jnp.float32),
            pltpu.VMEM((32, 128), jnp.float32),
            pltpu.VMEM((32, 128), jnp.float32),
            pltpu.VMEM((32, 128), jnp.float32),
            pltpu.SemaphoreType.DMA,
            pltpu.SemaphoreType.DMA,
            pltpu.SemaphoreType.DMA,
            pltpu.SemaphoreType.DMA,
        ],
    )
    def tr(tblT_hbm, tail_hbm, s128_hbm, in0, in1, ot0, ot1, gi0, gi1, go0, go1):
        wid = lax.axis_index("s") * _NC + lax.axis_index("c")
        ins = (in0, in1)
        ots = (ot0, ot1)
        gis = (gi0, gi1)
        gos = (go0, go1)
        lane = lax.iota(jnp.int32, 16)

        def start_in(t, b):
            bid = t * _NW + wid
            @pl.when(bid < _CBF)
            def _():
                pltpu.async_copy(
                    tblT_hbm.at[:, pl.ds(bid * 128, 128)], ins[b], gis[b])

        def wait_in(b):
            pltpu.make_async_copy(
                tblT_hbm.at[:, pl.ds(0, 128)], ins[b], gis[b]).wait()

        def start_out(b, bid):
            pltpu.async_copy(
                ots[b], s128_hbm.at[pl.ds(bid * 32, 32), :], gos[b])

        def wait_out(b):
            pltpu.make_async_copy(
                ots[b], s128_hbm.at[pl.ds(0, 32), :], gos[b]).wait()

        def shuffle(b, nrow):
            @plsc.parallel_loop(0, nrow, unroll=2)
            def _(r):
                vs = []
                for m in range(8):
                    rows = lane + 16 * (m % 2)
                    cols = jnp.full((16,), jnp.int32(4) * r + m // 2, jnp.int32)
                    vs.append(plsc.load_gather(ins[b], [rows, cols]))
                for m in range(8):
                    ots[b][r, pl.ds(16 * m, 16)] = vs[m]

        def body(u, carry):
            for dt in (0, 1):
                t = 2 * u + dt
                b = dt
                bid = t * _NW + wid

                @pl.when(bid < _CBF)
                def _():
                    start_in(t + 1, 1 - b)
                    wait_in(b)
                    @pl.when(t >= 2)
                    def _():
                        wait_out(b)
                    shuffle(b, 32)
                    start_out(b, bid)
            return carry

        start_in(0, 0)
        lax.fori_loop(0, (_TPW + 1) // 2, body, 0)
        for dt in (0, 1):
            t = _TPW - 2 + dt
            bid = t * _NW + wid
            @pl.when(bid < _CBF)
            def _():
                wait_out(t % 2)

        @pl.when(wid == 0)
        def _():
            pltpu.sync_copy(tail_hbm, ot0.at[pl.ds(0, 16), :])
            pltpu.sync_copy(ot0.at[pl.ds(0, 16), :],
                            s128_hbm.at[pl.ds(_CBF * 32, 16), :])

    return tr


def _make_gather():
    mesh = plsc.VectorSubcoreMesh(core_axis_name="c", subcore_axis_name="s")

    @functools.partial(
        pl.kernel,
        mesh=mesh,
        out_type=jax.ShapeDtypeStruct((_N_FIELDS, _D, _BATCH), jnp.float32),
        compiler_params=pltpu.CompilerParams(needs_layout_passes=False),
        scratch_types=[
            pltpu.VMEM((_N_FIELDS, _BPW), jnp.int32),
            pltpu.VMEM((128,), jnp.int32),
            pltpu.VMEM((128,), jnp.int32),
            pltpu.VMEM((128, 128), jnp.float32),
            pltpu.VMEM((128, 128), jnp.float32),
            pltpu.VMEM((32, 128), jnp.float32),
            pltpu.VMEM((32, 128), jnp.float32),
            pltpu.SemaphoreType.DMA,
            pltpu.SemaphoreType.DMA,
            pltpu.SemaphoreType.DMA,
            pltpu.SemaphoreType.DMA,
        ],
    )
    def ga(idxT_hbm, s128_hbm, out3_hbm, idx_v, ri0, ri1, rv0, rv1, ot0, ot1,
           gsem0, gsem1, osem0, osem1):
        wid = lax.axis_index("s") * _NC + lax.axis_index("c")
        b0 = wid * _BPW
        pltpu.sync_copy(idxT_hbm.at[:, pl.ds(b0, _BPW)], idx_v)
        ris = (ri0, ri1)
        rvs = (rv0, rv1)
        ots = (ot0, ot1)
        gsems = (gsem0, gsem1)
        osems = (osem0, osem1)
        lane = lax.iota(jnp.int32, 16)

        def prep_and_gather(g, b):
            f = g // 4
            tb = g % 4
            ivs = [idx_v[f, pl.ds(tb * 128 + 16 * m, 16)] for m in range(8)]
            for m in range(8):
                ris[b][pl.ds(16 * m, 16)] = ivs[m] >> 2
            pltpu.async_copy(s128_hbm.at[ris[b]], rvs[b], gsems[b])

        def wait_gather(b):
            pltpu.make_async_copy(s128_hbm.at[ris[b]], rvs[b], gsems[b]).wait()

        def extract(g, b):
            f = g // 4
            tb = g % 4
            avecs = []
            for m in range(8):
                iv = idx_v[f, pl.ds(tb * 128 + 16 * m, 16)]
                avecs.append((iv & 3) << 5)
            @plsc.parallel_loop(0, 32, unroll=4)
            def _(j):
                vs = []
                for m in range(8):
                    vs.append(plsc.load_gather(
                        rvs[b], [lane + 16 * m, avecs[m] + j]))
                for m in range(8):
                    ots[b][j, pl.ds(16 * m, 16)] = vs[m]

        def start_out(g, b):
            f = g // 4
            tb = g % 4
            pltpu.async_copy(
                ots[b], out3_hbm.at[f, :, pl.ds((4 * wid + tb) * 128, 128)],
                osems[b])

        def wait_out(b):
            pltpu.make_async_copy(
                ots[b], out3_hbm.at[0, :, pl.ds(0, 128)], osems[b]).wait()

        def body(u, carry):
            for dt in (0, 1):
                g = 2 * u + dt
                b = dt
                @pl.when(g + 1 < _NGRP)
                def _():
                    prep_and_gather(g + 1, 1 - b)
                wait_gather(b)
                @pl.when(g >= 2)
                def _():
                    wait_out(b)
                extract(g, b)
                start_out(g, b)
            return carry

        prep_and_gather(0, 0)
        lax.fori_loop(0, _NGRP // 2, body, 0)
        wait_out(0)
        wait_out(1)

    return ga


_tr = _make_transpose()
_ga = _make_gather()


def kernel(x, table):
    tail = table[_CBF * 128:].reshape(16, 128)
    s128 = _tr(table.T, tail)
    out3 = _ga(x.T, s128)
    return out3.transpose(2, 0, 1)
